# baseline (speedup 1.0000x reference)
import jax
import jax.numpy as jnp
from jax.experimental import pallas as pl
from jax.experimental.pallas import tpu as pltpu

N, H, F, D = 16384, 1024, 512, 512
BN = 1024
NBLK = N // BN
NBUF = 2


def _outer(pf_hbm, pm_hbm, nf_hbm, nm_hbm, bm_hbm,
           ws_ref, bs_ref, wh_ref, bh_ref, wsc_ref, bsc_ref,
           out_ref):
    out_ref[...] = jnp.zeros_like(out_ref)

    def inner(pf_ref, pm_ref, nf_ref, nm_ref, bm_ref):
        b = bs_ref[...] + bh_ref[...]
        ws = ws_ref[...]
        wh = wh_ref[...]
        wsc_row = wsc_ref[...].reshape(1, D)

        xp = jnp.dot(pf_ref[...], ws, preferred_element_type=jnp.float32)
        xp = xp + jnp.dot(pm_ref[...], wh, preferred_element_type=jnp.float32) + b
        sp = jnp.sum(jnp.maximum(xp, 0.0) * wsc_row, axis=1, keepdims=True)

        xn = jnp.dot(nf_ref[...], ws, preferred_element_type=jnp.float32)
        xn = xn + jnp.dot(nm_ref[...], wh, preferred_element_type=jnp.float32) + b
        sn = jnp.sum(jnp.maximum(xn, 0.0) * wsc_row, axis=1, keepdims=True)

        ones = jnp.ones((BN, 1), jnp.float32)
        svec = jnp.concatenate([sp, sn, ones, ones], axis=1)

        contrib = jax.lax.dot_general(
            bm_ref[...], svec, (((0,), (0,)), ((), ())),
            preferred_element_type=jnp.float32)
        out_ref[...] += contrib

    buf = pl.Buffered(buffer_count=NBUF)
    row_spec = pl.BlockSpec((BN, F), lambda i: (i, 0), pipeline_mode=buf)
    mask_spec = pl.BlockSpec((BN, H), lambda i: (i, 0), pipeline_mode=buf)

    pipeline = pltpu.emit_pipeline(
        inner,
        grid=(NBLK,),
        in_specs=[row_spec, row_spec, row_spec, row_spec, mask_spec],
    )
    pipeline(pf_hbm, pm_hbm, nf_hbm, nm_hbm, bm_hbm)

    acc = out_ref[...]
    colsum = acc[:, 2:3]
    bsc = bsc_ref[0, 0]
    pos = jax.nn.sigmoid(acc[:, 0:1] / colsum + bsc)
    neg = jax.nn.sigmoid(acc[:, 1:2] / colsum + bsc)
    out_ref[...] = jnp.concatenate([pos, neg, colsum, colsum], axis=1)


@jax.jit
def kernel(pos_features, pos_matrix, neg_features, neg_matrix, batch_mask,
           W_self, b_self, W_hyper, b_hyper, W_score, b_score):
    pf = pos_features[0]
    pm = pos_matrix[0]
    nf = neg_features[0]
    nm = neg_matrix[0]
    bm = batch_mask[0]
    bs = b_self.reshape(1, D)
    bh = b_hyper.reshape(1, D)
    bsc = b_score.reshape(1, 1)

    hbm = pl.BlockSpec(memory_space=pl.ANY)
    vmem = pl.BlockSpec(memory_space=pltpu.VMEM)

    out = pl.pallas_call(
        _outer,
        in_specs=[hbm, hbm, hbm, hbm, hbm,
                  vmem, vmem, vmem, vmem, vmem, vmem],
        out_specs=pl.BlockSpec(memory_space=pltpu.VMEM),
        out_shape=jax.ShapeDtypeStruct((H, 4), jnp.float32),
    )(pf, pm, nf, nm, bm, W_self, bs, W_hyper, bh, W_score, bsc)

    return (out[:, 0:1], out[:, 1:2])


# speedup vs baseline: 1.0135x; 1.0135x over previous
import jax
import jax.numpy as jnp
from jax.experimental import pallas as pl
from jax.experimental.pallas import tpu as pltpu

N, H, F, D = 16384, 1024, 512, 512
BN = 1024
NBLK = N // BN
NBUF = 4


def _outer(pf_hbm, pm_hbm, nf_hbm, nm_hbm, bm_hbm,
           ws_ref, bs_ref, wh_ref, bh_ref, wsc_ref, bsc_ref,
           out_ref):
    out_ref[...] = jnp.zeros_like(out_ref)

    def inner(pf_ref, pm_ref, nf_ref, nm_ref, bm_ref):
        b = bs_ref[...] + bh_ref[...]
        ws = ws_ref[...]
        wh = wh_ref[...]
        wsc_row = wsc_ref[...].reshape(1, D)

        xp = jnp.dot(pf_ref[...], ws, preferred_element_type=jnp.float32)
        xp = xp + jnp.dot(pm_ref[...], wh, preferred_element_type=jnp.float32) + b
        sp = jnp.sum(jnp.maximum(xp, 0.0) * wsc_row, axis=1, keepdims=True)

        xn = jnp.dot(nf_ref[...], ws, preferred_element_type=jnp.float32)
        xn = xn + jnp.dot(nm_ref[...], wh, preferred_element_type=jnp.float32) + b
        sn = jnp.sum(jnp.maximum(xn, 0.0) * wsc_row, axis=1, keepdims=True)

        ones = jnp.ones((BN, 1), jnp.float32)
        svec = jnp.concatenate([sp, sn, ones, ones], axis=1)

        contrib = jax.lax.dot_general(
            bm_ref[...], svec, (((0,), (0,)), ((), ())),
            preferred_element_type=jnp.float32)
        out_ref[...] += contrib

    row_spec = pl.BlockSpec((BN, F), lambda i: (i, 0),
                            pipeline_mode=pl.Buffered(buffer_count=3))
    mask_spec = pl.BlockSpec((BN, H), lambda i: (i, 0),
                             pipeline_mode=pl.Buffered(buffer_count=4))

    pipeline = pltpu.emit_pipeline(
        inner,
        grid=(NBLK,),
        in_specs=[row_spec, row_spec, row_spec, row_spec, mask_spec],
    )
    pipeline(pf_hbm, pm_hbm, nf_hbm, nm_hbm, bm_hbm)

    acc = out_ref[...]
    colsum = acc[:, 2:3]
    bsc = bsc_ref[0, 0]
    pos = jax.nn.sigmoid(acc[:, 0:1] / colsum + bsc)
    neg = jax.nn.sigmoid(acc[:, 1:2] / colsum + bsc)
    out_ref[...] = jnp.concatenate([pos, neg, colsum, colsum], axis=1)


@jax.jit
def kernel(pos_features, pos_matrix, neg_features, neg_matrix, batch_mask,
           W_self, b_self, W_hyper, b_hyper, W_score, b_score):
    pf = pos_features[0]
    pm = pos_matrix[0]
    nf = neg_features[0]
    nm = neg_matrix[0]
    bm = batch_mask[0]
    bs = b_self.reshape(1, D)
    bh = b_hyper.reshape(1, D)
    bsc = b_score.reshape(1, 1)

    hbm = pl.BlockSpec(memory_space=pl.ANY)
    vmem = pl.BlockSpec(memory_space=pltpu.VMEM)

    out = pl.pallas_call(
        _outer,
        in_specs=[hbm, hbm, hbm, hbm, hbm,
                  vmem, vmem, vmem, vmem, vmem, vmem],
        out_specs=pl.BlockSpec(memory_space=pltpu.VMEM),
        out_shape=jax.ShapeDtypeStruct((H, 4), jnp.float32),
    )(pf, pm, nf, nm, bm, W_self, bs, W_hyper, bh, W_score, bsc)

    return (out[:, 0:1], out[:, 1:2])


# speedup vs baseline: 1.0263x; 1.0127x over previous
import jax
import jax.numpy as jnp
from jax.experimental import pallas as pl
from jax.experimental.pallas import tpu as pltpu

N, H, F, D = 16384, 1024, 512, 512
BN = 1024
NBLK = N // BN
NBUF = 3


def _outer(pf_hbm, pm_hbm, nf_hbm, nm_hbm, bm_hbm,
           ws_ref, bs_ref, wh_ref, bh_ref, wsc_ref, bsc_ref,
           out_ref):
    out_ref[...] = jnp.zeros_like(out_ref)

    def inner(pf_ref, pm_ref, nf_ref, nm_ref, bm_ref):
        b = bs_ref[...] + bh_ref[...]
        ws = ws_ref[...]
        wh = wh_ref[...]
        wsc_row = wsc_ref[...].reshape(1, D)

        xp = jnp.dot(pf_ref[...], ws, preferred_element_type=jnp.float32)
        xp = xp + jnp.dot(pm_ref[...], wh, preferred_element_type=jnp.float32) + b
        sp = jnp.sum(jnp.maximum(xp, 0.0) * wsc_row, axis=1, keepdims=True)

        xn = jnp.dot(nf_ref[...], ws, preferred_element_type=jnp.float32)
        xn = xn + jnp.dot(nm_ref[...], wh, preferred_element_type=jnp.float32) + b
        sn = jnp.sum(jnp.maximum(xn, 0.0) * wsc_row, axis=1, keepdims=True)

        ones = jnp.ones((BN, 1), jnp.float32)
        svec = jnp.concatenate([sp, sn, ones, ones], axis=1)

        contrib = jax.lax.dot_general(
            bm_ref[...], svec, (((0,), (0,)), ((), ())),
            preferred_element_type=jnp.float32)
        out_ref[...] += contrib

    buf = pl.Buffered(buffer_count=NBUF)
    row_spec = pl.BlockSpec((BN, F), lambda i: (i, 0), pipeline_mode=buf)
    mask_spec = pl.BlockSpec((BN, H), lambda i: (i, 0), pipeline_mode=buf)

    pipeline = pltpu.emit_pipeline(
        inner,
        grid=(NBLK,),
        in_specs=[row_spec, row_spec, row_spec, row_spec, mask_spec],
    )
    pipeline(pf_hbm, pm_hbm, nf_hbm, nm_hbm, bm_hbm)

    acc = out_ref[...]
    colsum = acc[:, 2:3]
    bsc = bsc_ref[0, 0]
    pos = jax.nn.sigmoid(acc[:, 0:1] / colsum + bsc)
    neg = jax.nn.sigmoid(acc[:, 1:2] / colsum + bsc)
    out_ref[...] = jnp.concatenate([pos, neg, colsum, colsum], axis=1)


@jax.jit
def kernel(pos_features, pos_matrix, neg_features, neg_matrix, batch_mask,
           W_self, b_self, W_hyper, b_hyper, W_score, b_score):
    pf = pos_features[0]
    pm = pos_matrix[0]
    nf = neg_features[0]
    nm = neg_matrix[0]
    bm = batch_mask[0]
    bs = b_self.reshape(1, D)
    bh = b_hyper.reshape(1, D)
    bsc = b_score.reshape(1, 1)

    hbm = pl.BlockSpec(memory_space=pl.ANY)
    vmem = pl.BlockSpec(memory_space=pltpu.VMEM)

    out = pl.pallas_call(
        _outer,
        in_specs=[hbm, hbm, hbm, hbm, hbm,
                  vmem, vmem, vmem, vmem, vmem, vmem],
        out_specs=pl.BlockSpec(memory_space=pltpu.VMEM),
        out_shape=jax.ShapeDtypeStruct((H, 4), jnp.float32),
    )(pf, pm, nf, nm, bm, W_self, bs, W_hyper, bh, W_score, bsc)

    return (out[:, 0:1], out[:, 1:2])
